# Initial kernel scaffold; baseline (speedup 1.0000x reference)
import jax
import jax.numpy as jnp
from jax.experimental import pallas as pl


def kernel(desc1, desc2):
    raise NotImplementedError("write your pallas kernel here")


# speedup vs baseline: 1.0487x; 1.0487x over previous
import functools

import jax
import jax.numpy as jnp
from jax.experimental import pallas as pl

BM = 1024
BN = 2048


def _nn_kernel(a_ref, b_ref, dist_ref, idx_ref, *, n_blocks):
    j = pl.program_id(1)

    a = a_ref[...]
    b = b_ref[...]
    x = jax.lax.dot_general(
        a, b, (((1,), (1,)), ((), ())), preferred_element_type=jnp.float32
    )
    a2 = jnp.sum(a * a, axis=1)
    b2 = jnp.sum(b * b, axis=1)
    d2 = jnp.maximum(a2[:, None] + b2[None, :] - 2.0 * x, 0.0)

    local_min = jnp.min(d2, axis=1)
    local_arg = jnp.argmin(d2, axis=1).astype(jnp.int32) + j * BN

    @pl.when(j == 0)
    def _init():
        dist_ref[...] = local_min[:, None]
        idx_ref[...] = local_arg[:, None]

    @pl.when(j > 0)
    def _merge():
        prev_min = dist_ref[:, 0]
        prev_arg = idx_ref[:, 0]
        better = local_min < prev_min
        dist_ref[...] = jnp.where(better, local_min, prev_min)[:, None]
        idx_ref[...] = jnp.where(better, local_arg, prev_arg)[:, None]

    @pl.when(j == n_blocks - 1)
    def _finish():
        dist_ref[...] = jnp.sqrt(dist_ref[...])


def kernel(desc1, desc2):
    m, k = desc1.shape
    n, _ = desc2.shape
    m_blocks = m // BM
    n_blocks = n // BN

    dists, idxs = pl.pallas_call(
        functools.partial(_nn_kernel, n_blocks=n_blocks),
        grid=(m_blocks, n_blocks),
        in_specs=[
            pl.BlockSpec((BM, k), lambda i, j: (i, 0)),
            pl.BlockSpec((BN, k), lambda i, j: (j, 0)),
        ],
        out_specs=[
            pl.BlockSpec((BM, 1), lambda i, j: (i, 0)),
            pl.BlockSpec((BM, 1), lambda i, j: (i, 0)),
        ],
        out_shape=[
            jax.ShapeDtypeStruct((m, 1), jnp.float32),
            jax.ShapeDtypeStruct((m, 1), jnp.int32),
        ],
    )(desc1, desc2)

    idxs_in_1 = jnp.arange(m, dtype=jnp.int32).reshape(-1, 1)
    matches_idxs = jnp.concatenate([idxs_in_1, idxs], axis=1)
    return (dists, matches_idxs)


# speedup vs baseline: 2.0400x; 1.9452x over previous
import functools

import jax
import jax.numpy as jnp
from jax.experimental import pallas as pl

BM = 1024
BN = 2048
LANES = 128


def _nn_kernel(a_ref, b_ref, dist_ref, idx_ref, *, n_blocks):
    j = pl.program_id(1)

    a = a_ref[...]
    b = b_ref[...]
    x = jax.lax.dot_general(
        a * -2.0, b, (((1,), (1,)), ((), ())), preferred_element_type=jnp.float32
    )
    b2 = jnp.sum(b * b, axis=1)

    n_chunks = BN // LANES
    m = x[:, 0:LANES] + b2[0:LANES][None, :]
    kk = jnp.zeros((BM, LANES), jnp.int32)
    for t in range(1, n_chunks):
        c = x[:, t * LANES:(t + 1) * LANES] + b2[t * LANES:(t + 1) * LANES][None, :]
        better = c < m
        m = jnp.where(better, c, m)
        kk = jnp.where(better, t, kk)

    lane_arg = jnp.argmin(m, axis=1).astype(jnp.int32)
    local_min = jnp.min(m, axis=1)
    onehot = jax.lax.broadcasted_iota(jnp.int32, (BM, LANES), 1) == lane_arg[:, None]
    chunk = jnp.max(jnp.where(onehot, kk, 0), axis=1)
    local_arg = chunk * LANES + lane_arg + j * BN

    @pl.when(j == 0)
    def _init():
        dist_ref[...] = local_min[:, None]
        idx_ref[...] = local_arg[:, None]

    @pl.when(j > 0)
    def _merge():
        prev_min = dist_ref[:, 0]
        prev_arg = idx_ref[:, 0]
        better = local_min < prev_min
        dist_ref[...] = jnp.where(better, local_min, prev_min)[:, None]
        idx_ref[...] = jnp.where(better, local_arg, prev_arg)[:, None]

    @pl.when(j == n_blocks - 1)
    def _finish():
        a2 = jnp.sum(a * a, axis=1)[:, None]
        dist_ref[...] = jnp.sqrt(jnp.maximum(dist_ref[...] + a2, 0.0))


def kernel(desc1, desc2):
    m, k = desc1.shape
    n, _ = desc2.shape
    m_blocks = m // BM
    n_blocks = n // BN

    dists, idxs = pl.pallas_call(
        functools.partial(_nn_kernel, n_blocks=n_blocks),
        grid=(m_blocks, n_blocks),
        in_specs=[
            pl.BlockSpec((BM, k), lambda i, j: (i, 0)),
            pl.BlockSpec((BN, k), lambda i, j: (j, 0)),
        ],
        out_specs=[
            pl.BlockSpec((BM, 1), lambda i, j: (i, 0)),
            pl.BlockSpec((BM, 1), lambda i, j: (i, 0)),
        ],
        out_shape=[
            jax.ShapeDtypeStruct((m, 1), jnp.float32),
            jax.ShapeDtypeStruct((m, 1), jnp.int32),
        ],
    )(desc1, desc2)

    idxs_in_1 = jnp.arange(m, dtype=jnp.int32).reshape(-1, 1)
    matches_idxs = jnp.concatenate([idxs_in_1, idxs], axis=1)
    return (dists, matches_idxs)
